# baseline (speedup 1.0000x reference)
import jax
import jax.numpy as jnp
from jax import lax
from jax.experimental import pallas as pl
from jax.experimental.pallas import tpu as pltpu
from jax.experimental.pallas import tpu_sc as plsc

VOCAB = 1000000
MAX_LEN = 200
FEAT = 64
BATCH = 4096

NC = 2
NS = 16
NW = NC * NS
SEQ_PER_W = BATCH // NW
LANES = 16
VPF = FEAT // LANES


def _emb_body(x_hbm, emb_hbm, pe_hbm, out_hbm, idx_v, rows_v, pe_v, gsem):
    wid = lax.axis_index("s") * NC + lax.axis_index("c")
    base = wid * SEQ_PER_W
    pltpu.sync_copy(pe_hbm, pe_v)

    def seq_body(s, _):
        seq = base + s
        pltpu.sync_copy(x_hbm.at[seq], idx_v)
        cp1 = pltpu.async_copy(
            emb_hbm.at[idx_v.at[pl.ds(0, 128)]], rows_v.at[pl.ds(0, 128)], gsem)
        cp2 = pltpu.async_copy(
            emb_hbm.at[idx_v.at[pl.ds(128, 72)]], rows_v.at[pl.ds(128, 72)], gsem)
        cp1.wait()
        cp2.wait()

        def add_rows(r, _):
            for u in range(2):
                for c in range(VPF):
                    sl = pl.ds(c * LANES, LANES)
                    rows_v[2 * r + u, sl] = rows_v[2 * r + u, sl] + pe_v[2 * r + u, sl]
            return 0

        lax.fori_loop(0, MAX_LEN // 2, add_rows, 0)
        pltpu.sync_copy(rows_v, out_hbm.at[seq])
        return 0

    lax.fori_loop(0, SEQ_PER_W, seq_body, 0)


def kernel(x, emb_token, pe):
    x = x.astype(jnp.int32)
    mesh = plsc.VectorSubcoreMesh(core_axis_name="c", subcore_axis_name="s")
    return pl.kernel(
        _emb_body,
        out_type=jax.ShapeDtypeStruct((BATCH, MAX_LEN, FEAT), jnp.float32),
        mesh=mesh,
        compiler_params=pltpu.CompilerParams(use_tc_tiling_on_sc=False),
        scratch_types=[
            pltpu.VMEM((MAX_LEN,), jnp.int32),
            pltpu.VMEM((MAX_LEN, FEAT), jnp.float32),
            pltpu.VMEM((MAX_LEN, FEAT), jnp.float32),
            pltpu.SemaphoreType.DMA,
        ],
    )(x, emb_token, pe)


# speedup vs baseline: 1.1675x; 1.1675x over previous
import jax
import jax.numpy as jnp
from jax import lax
from jax.experimental import pallas as pl
from jax.experimental.pallas import tpu as pltpu
from jax.experimental.pallas import tpu_sc as plsc

VOCAB = 1000000
MAX_LEN = 200
FEAT = 64
BATCH = 4096

NC = 2
NS = 16
NW = NC * NS
SEQ_PER_W = BATCH // NW
LANES = 16
VPF = FEAT // LANES
NBUF = 4
CH0, CH1 = 128, MAX_LEN - 128


def _emb_body(x_hbm, emb_hbm, pe_hbm, out_hbm, idx_all, rows, pe_v, gsem, osem):
    wid = lax.axis_index("s") * NC + lax.axis_index("c")
    base = wid * SEQ_PER_W
    pltpu.sync_copy(pe_hbm, pe_v)
    pltpu.sync_copy(x_hbm.at[pl.ds(base, SEQ_PER_W)], idx_all)

    def issue_gather(s, b):
        pltpu.async_copy(emb_hbm.at[idx_all.at[s, pl.ds(0, CH0)]],
                         rows.at[b, pl.ds(0, CH0)], gsem.at[b])
        pltpu.async_copy(emb_hbm.at[idx_all.at[s, pl.ds(CH0, CH1)]],
                         rows.at[b, pl.ds(CH0, CH1)], gsem.at[b])

    def wait_gather(s, b):
        pltpu.make_async_copy(emb_hbm.at[idx_all.at[s, pl.ds(0, CH0)]],
                              rows.at[b, pl.ds(0, CH0)], gsem.at[b]).wait()
        pltpu.make_async_copy(emb_hbm.at[idx_all.at[s, pl.ds(CH0, CH1)]],
                              rows.at[b, pl.ds(CH0, CH1)], gsem.at[b]).wait()

    def wait_store(b):
        pltpu.make_async_copy(rows.at[b], out_hbm.at[base], osem.at[b]).wait()

    for b in range(NBUF - 1):
        issue_gather(b, b)

    def outer(g, _):
        for b in range(NBUF):
            s = NBUF * g + b
            bf = (b + NBUF - 1) % NBUF
            f = s + NBUF - 1

            @pl.when(jnp.logical_and(s >= 1, f < SEQ_PER_W))
            def _():
                wait_store(bf)

            @pl.when(f < SEQ_PER_W)
            def _():
                issue_gather(f, bf)

            wait_gather(s, b)

            def add_rows(r, _):
                for u in range(2):
                    rr = 2 * r + u
                    for c in range(VPF):
                        sl = pl.ds(c * LANES, LANES)
                        rows[b, rr, sl] = rows[b, rr, sl] + pe_v[rr, sl]
                return 0

            lax.fori_loop(0, MAX_LEN // 2, add_rows, 0)
            pltpu.async_copy(rows.at[b], out_hbm.at[base + s], osem.at[b])
        return 0

    lax.fori_loop(0, SEQ_PER_W // NBUF, outer, 0)
    for b in range(NBUF):
        wait_store(b)


def kernel(x, emb_token, pe):
    x = x.astype(jnp.int32)
    mesh = plsc.VectorSubcoreMesh(core_axis_name="c", subcore_axis_name="s")
    return pl.kernel(
        _emb_body,
        out_type=jax.ShapeDtypeStruct((BATCH, MAX_LEN, FEAT), jnp.float32),
        mesh=mesh,
        compiler_params=pltpu.CompilerParams(use_tc_tiling_on_sc=False),
        scratch_types=[
            pltpu.VMEM((SEQ_PER_W, MAX_LEN), jnp.int32),
            pltpu.VMEM((NBUF, MAX_LEN, FEAT), jnp.float32),
            pltpu.VMEM((MAX_LEN, FEAT), jnp.float32),
            pltpu.SemaphoreType.DMA((NBUF,)),
            pltpu.SemaphoreType.DMA((NBUF,)),
        ],
    )(x, emb_token, pe)
